# baseline (speedup 1.0000x reference)
import functools

import jax
import jax.numpy as jnp
from jax import lax
from jax.experimental import pallas as pl
from jax.experimental.pallas import tpu as pltpu
from jax.experimental.pallas import tpu_sc as plsc

_N, _T, _C = 32, 577, 768
_R = 256
_K = _T - _R
_KP = 384

_NEG = float("-inf")

_QCHUNK = _KP // 4
_SCHUNK = 32
_ROUNDS = _N * 4 // 32


def _sel_body(q_ref, m_ref, ts_ref, sel_ref, tso_ref):
    t, c, r, kp = _T, _C, _R, _KP
    f32 = jnp.float32

    m = m_ref[0]
    q = q_ref[0]
    ts_c = ts_ref[0]

    iota_r = lax.broadcasted_iota(jnp.int32, (t, t), 0).astype(f32)
    iota_c = lax.broadcasted_iota(jnp.int32, (t, t), 1).astype(f32)

    norm = jnp.sqrt(jnp.sum(m * m, axis=-1, keepdims=True))
    mn = m / norm
    sim = lax.dot_general(
        mn, mn, (((1,), (1,)), ((), ())),
        preferred_element_type=f32, precision=lax.Precision.DEFAULT)
    protect = ((iota_r == 0.0) | (iota_r == t - 1.0)
               | (iota_c == 0.0) | (iota_c == t - 1.0) | (iota_r == iota_c))
    sim = jnp.where(protect, _NEG, sim)

    nm_c = jnp.max(sim, axis=1, keepdims=True)
    nm_r = jnp.transpose(nm_c)
    allowed = (nm_r < nm_c) | ((nm_r == nm_c) & (iota_c > iota_r))
    mm_c = jnp.max(jnp.where(allowed, sim, _NEG), axis=1, keepdims=True)

    q_last = q[:, c - 1:c]
    imp_c = (jnp.sum(q, axis=-1, keepdims=True) - q_last) / (c - 1) + q_last
    score_c = imp_c - mm_c
    score_r = jnp.transpose(score_c)
    imp_ts = jnp.transpose(jnp.concatenate([imp_c, ts_c], axis=1))
    imp_r = imp_ts[0:1, :]
    ts_r = imp_ts[1:2, :]

    cmp = (score_r < score_c) | ((score_r == score_c) & (iota_c < iota_r))
    rank_c = jnp.sum(cmp.astype(f32), axis=1, keepdims=True)
    rank_r = jnp.transpose(rank_c)
    is_src_c = rank_c < r
    is_dst_c = ~is_src_c
    is_dst_r = rank_r >= r

    pos_r = jnp.sum((is_dst_c & (iota_r < iota_c)).astype(f32), axis=0,
                    keepdims=True)

    simd = jnp.where(is_dst_r, sim, _NEG)
    cm_c = jnp.max(simd, axis=1, keepdims=True)
    choice_c = jnp.min(jnp.where(simd == cm_c, iota_c, f32(t)), axis=1,
                       keepdims=True)
    cmat = iota_c == choice_c

    cpos_c = jnp.sum(jnp.where(cmat, pos_r, 0.0), axis=1, keepdims=True)
    ichoice_c = jnp.sum(jnp.where(cmat, imp_r, 0.0), axis=1, keepdims=True)

    mx = jnp.maximum(imp_c, ichoice_c)
    es = jnp.exp(imp_c - mx)
    ed = jnp.exp(ichoice_c - mx)
    w0 = 2.0 * es / (es + ed)
    srcf_c = is_src_c.astype(f32)
    aw_c = srcf_c * w0
    bw_c = srcf_c * (1.0 - w0)
    tsrc_c = srcf_c * ts_c

    coefdiag_r = 1.0 + jnp.sum(jnp.where(cmat, bw_c, 0.0), axis=0,
                               keepdims=True)
    tst_r = ts_r + jnp.sum(jnp.where(cmat, tsrc_c, 0.0), axis=0,
                           keepdims=True)

    tst_choice_c = jnp.sum(jnp.where(cmat, tst_r, 0.0), axis=1, keepdims=True)
    asc_c = aw_c * ts_c / tst_choice_c
    cpos_asc = jnp.transpose(jnp.concatenate([cpos_c, asc_c], axis=1))
    cpos_r = cpos_asc[0:1, :]
    asc_r = cpos_asc[1:2, :]
    coefts_r = coefdiag_r * ts_r / tst_r

    iota_k = lax.broadcasted_iota(jnp.int32, (kp, t), 0).astype(f32)
    iota_kj = lax.broadcasted_iota(jnp.int32, (kp, t), 1).astype(f32)
    g = (pos_r == iota_k) & is_dst_r
    oh_src = (rank_r == iota_k) & (iota_k < r)

    dtok = jnp.sum(jnp.where(g, iota_kj, 0.0), axis=1, keepdims=True)
    dscale = jnp.sum(jnp.where(g, coefts_r, 0.0), axis=1, keepdims=True)
    stok = jnp.sum(jnp.where(oh_src, iota_kj, 0.0), axis=1, keepdims=True)
    sscale = jnp.sum(jnp.where(oh_src, asc_r, 0.0), axis=1, keepdims=True)
    stgt = jnp.sum(jnp.where(oh_src, cpos_r, 0.0), axis=1, keepdims=True)

    sel = jnp.transpose(jnp.concatenate(
        [dtok, dscale, stok, sscale, stgt], axis=1))
    sel_ref[0] = sel

    tstc = jnp.sum(jnp.where(g, tst_r, 0.0), axis=1, keepdims=True)
    tso_ref[0] = tstc[:_K, :]


def _tc_select(query, metric, token_size):
    n, t, c, k, kp = _N, _T, _C, _K, _KP
    return pl.pallas_call(
        _sel_body,
        grid=(n,),
        in_specs=[
            pl.BlockSpec((1, t, c), lambda b: (b, 0, 0)),
            pl.BlockSpec((1, t, c), lambda b: (b, 0, 0)),
            pl.BlockSpec((1, t, 1), lambda b: (b, 0, 0)),
        ],
        out_specs=[
            pl.BlockSpec((1, 5, kp), lambda b: (b, 0, 0)),
            pl.BlockSpec((1, k, 1), lambda b: (b, 0, 0)),
        ],
        out_shape=[
            jax.ShapeDtypeStruct((n, 5, kp), jnp.float32),
            jax.ShapeDtypeStruct((n, k, 1), jnp.float32),
        ],
        compiler_params=pltpu.CompilerParams(
            dimension_semantics=("arbitrary",)),
    )(query, metric, token_size)


def _sc_merge_body(x_hbm, sel_hbm, out_hbm, selbuf, idxd, idxs, xbuf, acc,
                   sem):
    cid = lax.axis_index("c")
    sid = lax.axis_index("s")
    wid = cid * 16 + sid
    quarter = wid % 4
    base_batch = wid // 4
    r0 = quarter * _QCHUNK
    lanes = 16
    ncol = _C // lanes

    def do_round(rr, _):
        batch = base_batch + 8 * rr
        pltpu.sync_copy(sel_hbm.at[batch], selbuf)

        for i in range(_QCHUNK // lanes):
            idxd[pl.ds(i * lanes, lanes)] = selbuf[
                0, pl.ds(r0 + i * lanes, lanes)].astype(jnp.int32)
        pltpu.async_copy(x_hbm.at[batch].at[idxd], acc, sem).wait()

        def dgrp(jj, _2):
            sv = selbuf[1, pl.ds(r0 + jj * lanes, lanes)]
            for ji in range(lanes):
                s = sv[ji]
                j = jj * lanes + ji

                def col(kk, _3):
                    sl = pl.ds(kk * lanes, lanes)
                    acc[j, sl] = acc[j, sl] * s
                    return _3
                lax.fori_loop(0, ncol, col, 0, unroll=8)
            return _2
        lax.fori_loop(0, _QCHUNK // lanes, dgrp, 0)

        def schunk(ch, _2):
            c0 = ch * _SCHUNK
            for i in range(_SCHUNK // lanes):
                idxs[pl.ds(i * lanes, lanes)] = selbuf[
                    2, pl.ds(c0 + i * lanes, lanes)].astype(jnp.int32)
            pltpu.async_copy(x_hbm.at[batch].at[idxs], xbuf, sem).wait()

            def sgrp(jj, _3):
                sv_s = selbuf[3, pl.ds(c0 + jj * lanes, lanes)]
                sv_t = selbuf[4, pl.ds(c0 + jj * lanes, lanes)].astype(
                    jnp.int32)
                for ji in range(lanes):
                    tgt = sv_t[ji]
                    valid = (tgt >= r0) & (tgt < r0 + _QCHUNK)
                    local = jnp.where(valid, tgt - r0, 0)
                    s = jnp.where(valid, sv_s[ji], 0.0)
                    j = jj * lanes + ji

                    def col(kk, _4):
                        sl = pl.ds(kk * lanes, lanes)
                        acc[local, sl] = acc[local, sl] + xbuf[j, sl] * s
                        return _4
                    lax.fori_loop(0, ncol, col, 0, unroll=8)
                return _3
            lax.fori_loop(0, _SCHUNK // lanes, sgrp, 0)
            return _2
        lax.fori_loop(0, _R // _SCHUNK, schunk, 0)

        @pl.when(quarter < 3)
        def _copy96():
            pltpu.sync_copy(acc, out_hbm.at[batch].at[pl.ds(r0, _QCHUNK)])

        @pl.when(quarter == 3)
        def _copy40():
            pltpu.sync_copy(acc.at[pl.ds(0, 40)],
                            out_hbm.at[batch].at[pl.ds(r0, 40)])
        return _

    lax.fori_loop(0, _ROUNDS, do_round, 0)


def _sc_merge(x, sel):
    n, c, kp = _N, _C, _KP
    mesh = plsc.VectorSubcoreMesh(core_axis_name="c", subcore_axis_name="s")
    f = pl.kernel(
        _sc_merge_body, mesh=mesh,
        out_type=jax.ShapeDtypeStruct((n, 328, c), jnp.float32),
        scratch_types=[
            pltpu.VMEM((5, kp), jnp.float32),
            pltpu.VMEM((_QCHUNK,), jnp.int32),
            pltpu.VMEM((_SCHUNK,), jnp.int32),
            pltpu.VMEM((_SCHUNK, c), jnp.float32),
            pltpu.VMEM((_QCHUNK, c), jnp.float32),
            pltpu.SemaphoreType.DMA,
        ],
    )
    return f(x, sel)


@jax.jit
def kernel(x, query, metric, token_size):
    sel, ts_out = _tc_select(query, metric, token_size)
    x_out = _sc_merge(x, sel)[:, :_K, :]
    return (x_out, ts_out)


# speedup vs baseline: 1.0904x; 1.0904x over previous
import functools

import jax
import jax.numpy as jnp
from jax import lax
from jax.experimental import pallas as pl
from jax.experimental.pallas import tpu as pltpu
from jax.experimental.pallas import tpu_sc as plsc

_N, _T, _C = 32, 577, 768
_R = 256
_K = _T - _R
_KP = 384

_NEG = float("-inf")

_QCHUNK = _KP // 4
_SCHUNK = 32
_ROUNDS = _N * 4 // 32


def _sel_body(q_ref, m_ref, ts_ref, sel_ref, tso_ref):
    t, c, r, kp = _T, _C, _R, _KP
    f32 = jnp.float32

    m = m_ref[0]
    q = q_ref[0]
    ts_c = ts_ref[0]

    iota_r = lax.broadcasted_iota(jnp.int32, (t, t), 0).astype(f32)
    iota_c = lax.broadcasted_iota(jnp.int32, (t, t), 1).astype(f32)

    norm = jnp.sqrt(jnp.sum(m * m, axis=-1, keepdims=True))
    mn = m / norm
    sim = lax.dot_general(
        mn, mn, (((1,), (1,)), ((), ())),
        preferred_element_type=f32, precision=lax.Precision.DEFAULT)
    protect = ((iota_r == 0.0) | (iota_r == t - 1.0)
               | (iota_c == 0.0) | (iota_c == t - 1.0) | (iota_r == iota_c))
    sim = jnp.where(protect, _NEG, sim)

    nm_c = jnp.max(sim, axis=1, keepdims=True)
    nm_r = jnp.transpose(nm_c)
    allowed = (nm_r < nm_c) | ((nm_r == nm_c) & (iota_c > iota_r))
    mm_c = jnp.max(jnp.where(allowed, sim, _NEG), axis=1, keepdims=True)

    q_last = q[:, c - 1:c]
    imp_c = (jnp.sum(q, axis=-1, keepdims=True) - q_last) / (c - 1) + q_last
    score_c = imp_c - mm_c
    score_r = jnp.transpose(score_c)
    imp_ts = jnp.transpose(jnp.concatenate([imp_c, ts_c], axis=1))
    imp_r = imp_ts[0:1, :]
    ts_r = imp_ts[1:2, :]

    cmp = (score_r < score_c) | ((score_r == score_c) & (iota_c < iota_r))
    rank_c = jnp.sum(cmp.astype(f32), axis=1, keepdims=True)
    rank_r = jnp.transpose(rank_c)
    is_src_c = rank_c < r
    is_dst_c = ~is_src_c
    is_dst_r = rank_r >= r

    pos_r = jnp.sum((is_dst_c & (iota_r < iota_c)).astype(f32), axis=0,
                    keepdims=True)

    simd = jnp.where(is_dst_r, sim, _NEG)
    cm_c = jnp.max(simd, axis=1, keepdims=True)
    choice_c = jnp.min(jnp.where(simd == cm_c, iota_c, f32(t)), axis=1,
                       keepdims=True)
    cmat = iota_c == choice_c

    cpos_c = jnp.sum(jnp.where(cmat, pos_r, 0.0), axis=1, keepdims=True)
    ichoice_c = jnp.sum(jnp.where(cmat, imp_r, 0.0), axis=1, keepdims=True)

    mx = jnp.maximum(imp_c, ichoice_c)
    es = jnp.exp(imp_c - mx)
    ed = jnp.exp(ichoice_c - mx)
    w0 = 2.0 * es / (es + ed)
    srcf_c = is_src_c.astype(f32)
    aw_c = srcf_c * w0
    bw_c = srcf_c * (1.0 - w0)
    tsrc_c = srcf_c * ts_c

    coefdiag_r = 1.0 + jnp.sum(jnp.where(cmat, bw_c, 0.0), axis=0,
                               keepdims=True)
    tst_r = ts_r + jnp.sum(jnp.where(cmat, tsrc_c, 0.0), axis=0,
                           keepdims=True)

    tst_choice_c = jnp.sum(jnp.where(cmat, tst_r, 0.0), axis=1, keepdims=True)
    asc_c = aw_c * ts_c / tst_choice_c
    coefts_r = coefdiag_r * ts_r / tst_r

    cpos_r0 = jnp.transpose(cpos_c)
    is_src_r = ~is_dst_r
    csr_c = jnp.sum((is_src_r & ((cpos_r0 < cpos_c)
                                 | ((cpos_r0 == cpos_c)
                                    & (iota_c < iota_r)))).astype(f32),
                    axis=1, keepdims=True)
    cpos_asc = jnp.transpose(jnp.concatenate([cpos_c, asc_c, csr_c], axis=1))
    cpos_r = cpos_asc[0:1, :]
    asc_r = cpos_asc[1:2, :]
    csr_r = cpos_asc[2:3, :]

    iota_k = lax.broadcasted_iota(jnp.int32, (kp, t), 0).astype(f32)
    iota_kj = lax.broadcasted_iota(jnp.int32, (kp, t), 1).astype(f32)
    g = (pos_r == iota_k) & is_dst_r
    oh_src = (csr_r == iota_k) & is_src_r

    dtok = jnp.sum(jnp.where(g, iota_kj, 0.0), axis=1, keepdims=True)
    dscale = jnp.sum(jnp.where(g, coefts_r, 0.0), axis=1, keepdims=True)
    stok = jnp.sum(jnp.where(oh_src, iota_kj, 0.0), axis=1, keepdims=True)
    sscale = jnp.sum(jnp.where(oh_src, asc_r, 0.0), axis=1, keepdims=True)
    stgt = jnp.sum(jnp.where(oh_src, cpos_r, 0.0), axis=1, keepdims=True)
    offs = jnp.sum((is_src_r & (cpos_r < iota_k * 96.0)).astype(f32),
                   axis=1, keepdims=True)

    sel = jnp.transpose(jnp.concatenate(
        [dtok, dscale, stok, sscale, stgt, offs], axis=1))
    sel_ref[0] = sel

    tstc = jnp.sum(jnp.where(g, tst_r, 0.0), axis=1, keepdims=True)
    tso_ref[0] = tstc[:_K, :]


def _tc_select(query, metric, token_size):
    n, t, c, k, kp = _N, _T, _C, _K, _KP
    return pl.pallas_call(
        _sel_body,
        grid=(n,),
        in_specs=[
            pl.BlockSpec((1, t, c), lambda b: (b, 0, 0)),
            pl.BlockSpec((1, t, c), lambda b: (b, 0, 0)),
            pl.BlockSpec((1, t, 1), lambda b: (b, 0, 0)),
        ],
        out_specs=[
            pl.BlockSpec((1, 6, kp), lambda b: (b, 0, 0)),
            pl.BlockSpec((1, k, 1), lambda b: (b, 0, 0)),
        ],
        out_shape=[
            jax.ShapeDtypeStruct((n, 6, kp), jnp.float32),
            jax.ShapeDtypeStruct((n, k, 1), jnp.float32),
        ],
        compiler_params=pltpu.CompilerParams(
            dimension_semantics=("arbitrary",)),
    )(query, metric, token_size)


def _sc_merge_body(x_hbm, sel_hbm, out_hbm, selbuf, idxd, idxs, xbuf, acc,
                   sem):
    cid = lax.axis_index("c")
    sid = lax.axis_index("s")
    wid = cid * 16 + sid
    base_batch = wid // 4
    lanes = 16
    ncol = _C // lanes

    def do_round(rr, _):
        batch = base_batch + 8 * rr
        quarter = (wid + rr) % 4
        r0 = quarter * _QCHUNK
        pltpu.sync_copy(sel_hbm.at[batch], selbuf)

        for i in range(_QCHUNK // lanes):
            idxd[pl.ds(i * lanes, lanes)] = selbuf[
                0, pl.ds(r0 + i * lanes, lanes)].astype(jnp.int32)
        pltpu.async_copy(x_hbm.at[batch].at[idxd], acc, sem).wait()

        def dgrp(jj, _2):
            sv = selbuf[1, pl.ds(r0 + jj * lanes, lanes)]
            for ji in range(lanes):
                s = sv[ji]
                j = jj * lanes + ji

                def col(kk, _3):
                    sl = pl.ds(kk * lanes, lanes)
                    acc[j, sl] = acc[j, sl] * s
                    return _3
                lax.fori_loop(0, ncol, col, 0, unroll=8)
            return _2
        lax.fori_loop(0, _QCHUNK // lanes, dgrp, 0)

        ovec = selbuf[5, pl.ds(0, lanes)]
        o = [ovec[i] for i in range(5)]
        startf = jnp.where(quarter == 0, o[0],
                           jnp.where(quarter == 1, o[1],
                                     jnp.where(quarter == 2, o[2], o[3])))
        endf = jnp.where(quarter == 0, o[1],
                         jnp.where(quarter == 1, o[2],
                                   jnp.where(quarter == 2, o[3], o[4])))
        start = startf.astype(jnp.int32)
        end = endf.astype(jnp.int32)
        nch = jnp.maximum((end + _SCHUNK - 1) // _SCHUNK, 0)

        def schunk(ch, _2):
            c0 = ch * _SCHUNK
            for i in range(_SCHUNK // lanes):
                idxs[pl.ds(i * lanes, lanes)] = selbuf[
                    2, pl.ds(c0 + i * lanes, lanes)].astype(jnp.int32)
            pltpu.async_copy(x_hbm.at[batch].at[idxs], xbuf, sem).wait()

            def sgrp(jj, _3):
                sv_s = selbuf[3, pl.ds(c0 + jj * lanes, lanes)]
                sv_t = selbuf[4, pl.ds(c0 + jj * lanes, lanes)].astype(
                    jnp.int32)
                for ji in range(lanes):
                    tgt = sv_t[ji]
                    valid = (tgt >= r0) & (tgt < r0 + _QCHUNK)
                    local = jnp.where(valid, tgt - r0, 0)
                    s = jnp.where(valid, sv_s[ji], 0.0)
                    j = jj * lanes + ji

                    def col(kk, _4):
                        sl = pl.ds(kk * lanes, lanes)
                        acc[local, sl] = acc[local, sl] + xbuf[j, sl] * s
                        return _4
                    lax.fori_loop(0, ncol, col, 0, unroll=8)
                return _3
            lax.fori_loop(0, _SCHUNK // lanes, sgrp, 0)
            return _2
        lax.fori_loop(0, nch, schunk, 0)

        @pl.when(quarter < 3)
        def _copy96():
            pltpu.sync_copy(acc, out_hbm.at[batch].at[pl.ds(r0, _QCHUNK)])

        @pl.when(quarter == 3)
        def _copy40():
            pltpu.sync_copy(acc.at[pl.ds(0, 40)],
                            out_hbm.at[batch].at[pl.ds(r0, 40)])
        return _

    lax.fori_loop(0, _ROUNDS, do_round, 0)


def _sc_merge(x, sel):
    n, c, kp = _N, _C, _KP
    mesh = plsc.VectorSubcoreMesh(core_axis_name="c", subcore_axis_name="s")
    f = pl.kernel(
        _sc_merge_body, mesh=mesh,
        out_type=jax.ShapeDtypeStruct((n, 328, c), jnp.float32),
        scratch_types=[
            pltpu.VMEM((6, kp), jnp.float32),
            pltpu.VMEM((_QCHUNK,), jnp.int32),
            pltpu.VMEM((_SCHUNK,), jnp.int32),
            pltpu.VMEM((_SCHUNK, c), jnp.float32),
            pltpu.VMEM((_QCHUNK, c), jnp.float32),
            pltpu.SemaphoreType.DMA,
        ],
    )
    return f(x, sel)


@jax.jit
def kernel(x, query, metric, token_size):
    sel, ts_out = _tc_select(query, metric, token_size)
    x_out = _sc_merge(x, sel)[:, :_K, :]
    return (x_out, ts_out)
